# baseline (speedup 1.0000x reference)
import jax
import jax.numpy as jnp
from jax.experimental import pallas as pl

B, N = 128, 100000
ROWS = 8
STEPS = B // ROWS
W = 512
NCHUNK = N // W
NFULL = NCHUNK * W
TAIL = N - NFULL

def _body(lg_ref, mk_ref, gm_ref, act_ref, lp_ref):
    NEG = jnp.float32(-1e9)
    EPS8 = jnp.float32(1e-8)
    EPS9 = jnp.float32(1e-9)

    def chunk(ref, j):
        return ref[:, pl.ds(pl.multiple_of(j * W, W), W)]

    lg_t = lg_ref[:, NFULL:N]
    mk_t = mk_ref[:, NFULL:N]
    gm_t = gm_ref[:, NFULL:N]
    ml_t = jnp.where(mk_t, lg_t, NEG)

    def p1(j, acc):
        return jnp.maximum(acc, jnp.where(chunk(mk_ref, j), chunk(lg_ref, j), NEG))

    accm = jax.lax.fori_loop(0, NCHUNK, p1, jnp.full((ROWS, W), NEG, jnp.float32))
    m = jnp.maximum(jnp.max(accm, axis=1), jnp.max(ml_t, axis=1))[:, None]

    def p2(j, acc):
        ml = jnp.where(chunk(mk_ref, j), chunk(lg_ref, j), NEG)
        return acc + jnp.exp(ml - m)

    accs = jax.lax.fori_loop(0, NCHUNK, p2, jnp.zeros((ROWS, W), jnp.float32))
    ssum = (jnp.sum(accs, axis=1) + jnp.sum(jnp.exp(ml_t - m), axis=1))[:, None]
    invc = jnp.where(m > jnp.float32(-0.5e9),
                     1.0 / (ssum * (1.0 + EPS8)), 0.0)

    def p3(j, carry):
        vm, ix, gl = carry
        gm = chunk(gm_ref, j)
        ml = jnp.where(chunk(mk_ref, j), chunk(lg_ref, j), NEG)
        v = jnp.log(jnp.exp(ml - m) * invc + EPS9) + gm
        upd = v > vm
        return (jnp.where(upd, v, vm),
                jnp.where(upd, jnp.int32(j), ix),
                jnp.where(upd, gm, gl))

    vm0 = jnp.full((ROWS, W), -jnp.inf, jnp.float32)
    vm, ix, gl = jax.lax.fori_loop(
        0, NCHUNK, p3,
        (vm0, jnp.zeros((ROWS, W), jnp.int32), jnp.zeros((ROWS, W), jnp.float32)))

    v_t = jnp.log(jnp.exp(ml_t - m) * invc + EPS9) + gm_t

    colc = jax.lax.broadcasted_iota(jnp.int32, (ROWS, W), 1)
    gcol = ix * W + colc
    vmax = jnp.maximum(jnp.max(vm, axis=1), jnp.max(v_t, axis=1))[:, None]
    big = jnp.int32(2 ** 30)
    iota_t = jax.lax.broadcasted_iota(jnp.int32, (ROWS, TAIL), 1) + NFULL
    cand = jnp.minimum(
        jnp.min(jnp.where(vm == vmax, gcol, big), axis=1),
        jnp.min(jnp.where(v_t == vmax, iota_t, big), axis=1))[:, None]
    g_at = (jnp.sum(jnp.where(gcol == cand, gl, 0.0), axis=1)
            + jnp.sum(jnp.where(iota_t == cand, gm_t, 0.0), axis=1))

    act_ref[0, 0, :] = cand[:, 0]
    lp_ref[0, 0, :] = vmax[:, 0] - g_at


def kernel(logits, mask, gumbel):
    acts, lps = pl.pallas_call(
        _body,
        grid=(STEPS,),
        in_specs=[
            pl.BlockSpec((ROWS, N), lambda i: (i, 0)),
            pl.BlockSpec((ROWS, N), lambda i: (i, 0)),
            pl.BlockSpec((ROWS, N), lambda i: (i, 0)),
        ],
        out_specs=[
            pl.BlockSpec((1, 1, ROWS), lambda i: (i, 0, 0)),
            pl.BlockSpec((1, 1, ROWS), lambda i: (i, 0, 0)),
        ],
        out_shape=[
            jax.ShapeDtypeStruct((STEPS, 1, ROWS), jnp.int32),
            jax.ShapeDtypeStruct((STEPS, 1, ROWS), jnp.float32),
        ],
    )(logits, mask, gumbel)
    return acts.reshape(B), lps.reshape(B)


# speedup vs baseline: 1.3261x; 1.3261x over previous
import jax
import jax.numpy as jnp
from jax.experimental import pallas as pl

B, N = 128, 100000
ROWS = 8
STEPS = B // ROWS
W = 512
K = 4
CW = W * K
NCHUNK = N // CW
NFULL = NCHUNK * CW
TAIL = N - NFULL


def _body(lg_ref, mk_ref, gm_ref, act_ref, lp_ref):
    NEG = jnp.float32(-1e9)
    EPS8 = jnp.float32(1e-8)
    EPS9 = jnp.float32(1e-9)

    def chunk(ref, j):
        return ref[:, pl.ds(pl.multiple_of(j * CW, CW), CW)]

    lg_t = lg_ref[:, NFULL:N]
    mk_t = mk_ref[:, NFULL:N]
    gm_t = gm_ref[:, NFULL:N]
    ml_t = jnp.where(mk_t, lg_t, NEG)

    def p1(j, acc):
        ml = jnp.where(chunk(mk_ref, j), chunk(lg_ref, j), NEG)
        for k in range(K):
            acc = jnp.maximum(acc, ml[:, k * W:(k + 1) * W])
        return acc

    accm = jax.lax.fori_loop(0, NCHUNK, p1, jnp.full((ROWS, W), NEG, jnp.float32))
    m = jnp.maximum(jnp.max(accm, axis=1), jnp.max(ml_t, axis=1))[:, None]

    def p2(j, acc):
        ml = jnp.where(chunk(mk_ref, j), chunk(lg_ref, j), NEG)
        e = jnp.exp(ml - m)
        for k in range(K):
            acc = acc + e[:, k * W:(k + 1) * W]
        return acc

    accs = jax.lax.fori_loop(0, NCHUNK, p2, jnp.zeros((ROWS, W), jnp.float32))
    ssum = (jnp.sum(accs, axis=1) + jnp.sum(jnp.exp(ml_t - m), axis=1))[:, None]
    invc = jnp.where(m > jnp.float32(-0.5e9),
                     1.0 / (ssum * (1.0 + EPS8)), 0.0)

    def p3(j, carry):
        vm, ix, gl = carry
        gm = chunk(gm_ref, j)
        ml = jnp.where(chunk(mk_ref, j), chunk(lg_ref, j), NEG)
        v = jnp.log(jnp.exp(ml - m) * invc + EPS9) + gm
        for k in range(K):
            vk = v[:, k * W:(k + 1) * W]
            gk = gm[:, k * W:(k + 1) * W]
            upd = vk > vm
            vm = jnp.where(upd, vk, vm)
            ix = jnp.where(upd, j * K + k, ix)
            gl = jnp.where(upd, gk, gl)
        return vm, ix, gl

    vm0 = jnp.full((ROWS, W), -jnp.inf, jnp.float32)
    vm, ix, gl = jax.lax.fori_loop(
        0, NCHUNK, p3,
        (vm0, jnp.zeros((ROWS, W), jnp.int32), jnp.zeros((ROWS, W), jnp.float32)))

    v_t = jnp.log(jnp.exp(ml_t - m) * invc + EPS9) + gm_t

    colc = jax.lax.broadcasted_iota(jnp.int32, (ROWS, W), 1)
    gcol = ix * W + colc
    vmax = jnp.maximum(jnp.max(vm, axis=1), jnp.max(v_t, axis=1))[:, None]
    big = jnp.int32(2 ** 30)
    iota_t = jax.lax.broadcasted_iota(jnp.int32, (ROWS, TAIL), 1) + NFULL
    cand = jnp.minimum(
        jnp.min(jnp.where(vm == vmax, gcol, big), axis=1),
        jnp.min(jnp.where(v_t == vmax, iota_t, big), axis=1))[:, None]
    g_at = (jnp.sum(jnp.where(gcol == cand, gl, 0.0), axis=1)
            + jnp.sum(jnp.where(iota_t == cand, gm_t, 0.0), axis=1))

    act_ref[0, 0, :] = cand[:, 0]
    lp_ref[0, 0, :] = vmax[:, 0] - g_at


def kernel(logits, mask, gumbel):
    acts, lps = pl.pallas_call(
        _body,
        grid=(STEPS,),
        in_specs=[
            pl.BlockSpec((ROWS, N), lambda i: (i, 0)),
            pl.BlockSpec((ROWS, N), lambda i: (i, 0)),
            pl.BlockSpec((ROWS, N), lambda i: (i, 0)),
        ],
        out_specs=[
            pl.BlockSpec((1, 1, ROWS), lambda i: (i, 0, 0)),
            pl.BlockSpec((1, 1, ROWS), lambda i: (i, 0, 0)),
        ],
        out_shape=[
            jax.ShapeDtypeStruct((STEPS, 1, ROWS), jnp.int32),
            jax.ShapeDtypeStruct((STEPS, 1, ROWS), jnp.float32),
        ],
    )(logits, mask, gumbel)
    return acts.reshape(B), lps.reshape(B)


# speedup vs baseline: 1.3272x; 1.0009x over previous
import jax
import jax.numpy as jnp
from jax.experimental import pallas as pl

B, N = 128, 100000
ROWS = 8
STEPS = B // ROWS
W = 512
K = 4
CW = W * K
NCHUNK = N // CW
NFULL = NCHUNK * CW
TAIL = N - NFULL


def _body(lg_ref, mk_ref, gm_ref, act_ref, lp_ref):
    NEG = jnp.float32(-1e9)
    EPS8 = jnp.float32(1e-8)
    EPS9 = jnp.float32(1e-9)

    def chunk(ref, j):
        return ref[:, pl.ds(pl.multiple_of(j * CW, CW), CW)]

    lg_t = lg_ref[:, NFULL:N]
    mk_t = mk_ref[:, NFULL:N]
    gm_t = gm_ref[:, NFULL:N]
    ml_t = jnp.where(mk_t != 0, lg_t, NEG)

    def p1(j, acc):
        ml = jnp.where(chunk(mk_ref, j) != 0, chunk(lg_ref, j), NEG)
        for k in range(K):
            acc = jnp.maximum(acc, ml[:, k * W:(k + 1) * W])
        return acc

    accm = jax.lax.fori_loop(0, NCHUNK, p1, jnp.full((ROWS, W), NEG, jnp.float32))
    m = jnp.maximum(jnp.max(accm, axis=1), jnp.max(ml_t, axis=1))[:, None]

    def p2(j, acc):
        ml = jnp.where(chunk(mk_ref, j) != 0, chunk(lg_ref, j), NEG)
        e = jnp.exp(ml - m)
        for k in range(K):
            acc = acc + e[:, k * W:(k + 1) * W]
        return acc

    accs = jax.lax.fori_loop(0, NCHUNK, p2, jnp.zeros((ROWS, W), jnp.float32))
    ssum = (jnp.sum(accs, axis=1) + jnp.sum(jnp.exp(ml_t - m), axis=1))[:, None]
    invc = jnp.where(m > jnp.float32(-0.5e9),
                     1.0 / (ssum * (1.0 + EPS8)), 0.0)

    def p3(j, carry):
        vm, ix, gl = carry
        gm = chunk(gm_ref, j)
        ml = jnp.where(chunk(mk_ref, j) != 0, chunk(lg_ref, j), NEG)
        v = jnp.log(jnp.exp(ml - m) * invc + EPS9) + gm
        for k in range(K):
            vk = v[:, k * W:(k + 1) * W]
            gk = gm[:, k * W:(k + 1) * W]
            upd = vk > vm
            vm = jnp.where(upd, vk, vm)
            ix = jnp.where(upd, j * K + k, ix)
            gl = jnp.where(upd, gk, gl)
        return vm, ix, gl

    vm0 = jnp.full((ROWS, W), -jnp.inf, jnp.float32)
    vm, ix, gl = jax.lax.fori_loop(
        0, NCHUNK, p3,
        (vm0, jnp.zeros((ROWS, W), jnp.int32), jnp.zeros((ROWS, W), jnp.float32)))

    v_t = jnp.log(jnp.exp(ml_t - m) * invc + EPS9) + gm_t

    colc = jax.lax.broadcasted_iota(jnp.int32, (ROWS, W), 1)
    gcol = ix * W + colc
    vmax = jnp.maximum(jnp.max(vm, axis=1), jnp.max(v_t, axis=1))[:, None]
    big = jnp.int32(2 ** 30)
    iota_t = jax.lax.broadcasted_iota(jnp.int32, (ROWS, TAIL), 1) + NFULL
    cand = jnp.minimum(
        jnp.min(jnp.where(vm == vmax, gcol, big), axis=1),
        jnp.min(jnp.where(v_t == vmax, iota_t, big), axis=1))[:, None]
    g_at = (jnp.sum(jnp.where(gcol == cand, gl, 0.0), axis=1)
            + jnp.sum(jnp.where(iota_t == cand, gm_t, 0.0), axis=1))

    act_ref[0, 0, :] = cand[:, 0]
    lp_ref[0, 0, :] = vmax[:, 0] - g_at


def kernel(logits, mask, gumbel):
    acts, lps = pl.pallas_call(
        _body,
        grid=(STEPS,),
        in_specs=[
            pl.BlockSpec((ROWS, N), lambda i: (i, 0)),
            pl.BlockSpec((ROWS, N), lambda i: (i, 0)),
            pl.BlockSpec((ROWS, N), lambda i: (i, 0)),
        ],
        out_specs=[
            pl.BlockSpec((1, 1, ROWS), lambda i: (i, 0, 0)),
            pl.BlockSpec((1, 1, ROWS), lambda i: (i, 0, 0)),
        ],
        out_shape=[
            jax.ShapeDtypeStruct((STEPS, 1, ROWS), jnp.int32),
            jax.ShapeDtypeStruct((STEPS, 1, ROWS), jnp.float32),
        ],
    )(logits, mask.astype(jnp.float32), gumbel)
    return acts.reshape(B), lps.reshape(B)


# speedup vs baseline: 2.7929x; 2.1043x over previous
import jax
import jax.numpy as jnp
from jax.experimental import pallas as pl
from jax.experimental.pallas import tpu as pltpu

B, N = 128, 100000
CH = 4000
NC = N // CH
SL = CH // 8


def _stats_body(lg_ref, mk_ref, m_ref, ic_ref, accm_ref, accs_ref):
    c = pl.program_id(0)
    NEG = jnp.float32(-1e9)

    @pl.when(c == 0)
    def _():
        accm_ref[...] = jnp.full((8, B), NEG, jnp.float32)
        accs_ref[...] = jnp.zeros((8, B), jnp.float32)

    ml3 = jnp.where(mk_ref[...] != 0, lg_ref[...], NEG).reshape(SL, 8, B)
    cm = jnp.max(ml3, axis=0)
    am = accm_ref[...]
    nm = jnp.maximum(am, cm)
    cs = jnp.sum(jnp.exp(ml3 - nm[None]), axis=0)
    ns = accs_ref[...] * jnp.exp(am - nm) + cs
    accm_ref[...] = nm
    accs_ref[...] = ns

    @pl.when(c == NC - 1)
    def _():
        m_b = jnp.max(nm, axis=0, keepdims=True)
        ssum = jnp.sum(ns * jnp.exp(nm - m_b), axis=0, keepdims=True)
        m_ref[...] = m_b
        ic_ref[...] = jnp.where(m_b > jnp.float32(-0.5e9),
                                1.0 / (ssum * (1.0 + jnp.float32(1e-8))), 0.0)


def _argmax_body(lg_ref, mk_ref, gm_ref, m_ref, ic_ref,
                 act_ref, lp_ref, vm_ref, ix_ref, gl_ref):
    c = pl.program_id(0)
    NEG = jnp.float32(-1e9)
    BIG = jnp.int32(2 ** 30)

    @pl.when(c == 0)
    def _():
        vm_ref[...] = jnp.full((8, B), -jnp.inf, jnp.float32)
        ix_ref[...] = jnp.zeros((8, B), jnp.int32)
        gl_ref[...] = jnp.zeros((8, B), jnp.float32)

    gm3 = gm_ref[...].reshape(SL, 8, B)
    ml3 = jnp.where(mk_ref[...] != 0, lg_ref[...], NEG).reshape(SL, 8, B)
    m_b = m_ref[...][None]
    ic = ic_ref[...][None]
    v3 = jnp.log(jnp.exp(ml3 - m_b) * ic + jnp.float32(1e-9)) + gm3

    cmv = jnp.max(v3, axis=0)
    i3 = jax.lax.broadcasted_iota(jnp.int32, (SL, 8, B), 0)
    ci = jnp.min(jnp.where(v3 == cmv[None], i3, BIG), axis=0)
    cg = jnp.sum(jnp.where(i3 == ci[None], gm3, 0.0), axis=0)

    sub = jax.lax.broadcasted_iota(jnp.int32, (8, B), 0)
    cr = c * CH + ci * 8 + sub

    vm = vm_ref[...]
    upd = cmv > vm
    nvm = jnp.where(upd, cmv, vm)
    nix = jnp.where(upd, cr, ix_ref[...])
    ngl = jnp.where(upd, cg, gl_ref[...])
    vm_ref[...] = nvm
    ix_ref[...] = nix
    gl_ref[...] = ngl

    @pl.when(c == NC - 1)
    def _():
        vmax = jnp.max(nvm, axis=0, keepdims=True)
        cand = jnp.min(jnp.where(nvm == vmax, nix, BIG), axis=0, keepdims=True)
        g_at = jnp.sum(jnp.where(nix == cand, ngl, 0.0), axis=0, keepdims=True)
        act_ref[...] = cand
        lp_ref[...] = vmax - g_at


def kernel(logits, mask, gumbel):
    lgt = logits.T
    gmt = gumbel.T
    mkt = mask.T.astype(jnp.uint8)

    m_b, ic_b = pl.pallas_call(
        _stats_body,
        grid=(NC,),
        in_specs=[
            pl.BlockSpec((CH, B), lambda c: (c, 0)),
            pl.BlockSpec((CH, B), lambda c: (c, 0)),
        ],
        out_specs=[
            pl.BlockSpec((1, B), lambda c: (0, 0)),
            pl.BlockSpec((1, B), lambda c: (0, 0)),
        ],
        out_shape=[
            jax.ShapeDtypeStruct((1, B), jnp.float32),
            jax.ShapeDtypeStruct((1, B), jnp.float32),
        ],
        scratch_shapes=[
            pltpu.VMEM((8, B), jnp.float32),
            pltpu.VMEM((8, B), jnp.float32),
        ],
    )(lgt, mkt)

    acts, lps = pl.pallas_call(
        _argmax_body,
        grid=(NC,),
        in_specs=[
            pl.BlockSpec((CH, B), lambda c: (c, 0)),
            pl.BlockSpec((CH, B), lambda c: (c, 0)),
            pl.BlockSpec((CH, B), lambda c: (c, 0)),
            pl.BlockSpec((1, B), lambda c: (0, 0)),
            pl.BlockSpec((1, B), lambda c: (0, 0)),
        ],
        out_specs=[
            pl.BlockSpec((1, B), lambda c: (0, 0)),
            pl.BlockSpec((1, B), lambda c: (0, 0)),
        ],
        out_shape=[
            jax.ShapeDtypeStruct((1, B), jnp.int32),
            jax.ShapeDtypeStruct((1, B), jnp.float32),
        ],
        scratch_shapes=[
            pltpu.VMEM((8, B), jnp.float32),
            pltpu.VMEM((8, B), jnp.int32),
            pltpu.VMEM((8, B), jnp.float32),
        ],
    )(lgt, mkt, gmt, m_b, ic_b)

    return acts.reshape(B), lps.reshape(B)
